# baseline (speedup 1.0000x reference)
import functools

import jax
import jax.numpy as jnp
from jax import lax
from jax.experimental import pallas as pl
from jax.experimental.pallas import tpu as pltpu
from jax.experimental.pallas import tpu_sc as plsc

N_SAMPLES = 100000
NPC_DIM = 128
BATCH = 4096
TEMP = 0.07
CONST = 1e-12

_NC = 2
_NS = 16
_NW = _NC * _NS
_BPW = BATCH // _NW

_TN = 1000
_NT = N_SAMPLES // _TN


@functools.cache
def _get_sc_gather():
    mesh = plsc.VectorSubcoreMesh(core_axis_name="c", subcore_axis_name="s")

    @functools.partial(
        pl.kernel,
        mesh=mesh,
        out_type=[
            jax.ShapeDtypeStruct((BATCH,), jnp.int32),
            jax.ShapeDtypeStruct((BATCH, NPC_DIM), jnp.float32),
            jax.ShapeDtypeStruct((BATCH, NPC_DIM), jnp.float32),
            jax.ShapeDtypeStruct((BATCH, NPC_DIM), jnp.float32),
        ],
        scratch_types=[
            pltpu.VMEM((_BPW,), jnp.int32),
            pltpu.VMEM((_BPW,), jnp.int32),
            pltpu.VMEM((_BPW,), jnp.int32),
            pltpu.VMEM((_BPW,), jnp.int32),
            pltpu.VMEM((_BPW, NPC_DIM), jnp.float32),
            pltpu.VMEM((_BPW, NPC_DIM), jnp.float32),
            pltpu.VMEM((_BPW, NPC_DIM), jnp.float32),
            pltpu.SemaphoreType.DMA,
        ],
    )
    def _sc_gather(index_hbm, lni_hbm, memory_hbm, flag_hbm, neighbors_hbm,
                   flags_out, rows_self_out, rows_ln_out, rows_nb_out,
                   idx_v, lni_v, nbi_v, flg_v, rs_v, rl_v, rn_v, sem):
        wid = lax.axis_index("s") * _NC + lax.axis_index("c")
        base = wid * _BPW
        pltpu.sync_copy(index_hbm.at[pl.ds(base, _BPW)], idx_v)
        pltpu.sync_copy(lni_hbm.at[pl.ds(base, _BPW)], lni_v)
        pltpu.async_copy(flag_hbm.at[idx_v], flg_v, sem).wait()
        pltpu.async_copy(neighbors_hbm.at[idx_v], nbi_v, sem).wait()
        pltpu.async_copy(memory_hbm.at[idx_v], rs_v, sem).wait()
        pltpu.async_copy(memory_hbm.at[lni_v], rl_v, sem).wait()
        pltpu.async_copy(memory_hbm.at[nbi_v], rn_v, sem).wait()
        pltpu.sync_copy(flg_v, flags_out.at[pl.ds(base, _BPW)])
        pltpu.sync_copy(rs_v, rows_self_out.at[pl.ds(base, _BPW)])
        pltpu.sync_copy(rl_v, rows_ln_out.at[pl.ds(base, _BPW)])
        pltpu.sync_copy(rn_v, rows_nb_out.at[pl.ds(base, _BPW)])

    return _sc_gather


_LOG2E = 1.4426950408889634


def _stream_body(zp_ref, mem_ref, znt_ref, se_ref, znt_bf_ref, acc_ref):
    step = pl.program_id(0)

    @pl.when(step == 0)
    def _init():
        zp = zp_ref[...]
        nrm = jnp.sqrt(jnp.sum(zp * zp, axis=1, keepdims=True))
        znt = zp * (_LOG2E / TEMP) / jnp.maximum(nrm, 1e-12)
        znt_ref[...] = znt
        znt_bf_ref[...] = znt.astype(jnp.bfloat16)
        acc_ref[...] = jnp.zeros_like(acc_ref)

    s = lax.dot_general(znt_bf_ref[...], mem_ref[...].astype(jnp.bfloat16),
                        (((1,), (1,)), ((), ())),
                        preferred_element_type=jnp.float32)
    e = jnp.exp2(s.astype(jnp.bfloat16))
    t = e[:, 0:128]
    for i in range(1, _TN // 128):
        t = t + e[:, i * 128:(i + 1) * 128]
    rem = _TN - (_TN // 128) * 128
    if rem:
        tail = e[:, _TN - rem:_TN]
        t = t + jnp.pad(tail, ((0, 0), (0, 128 - rem)))
    acc_ref[...] += t.astype(jnp.float32)

    @pl.when(step == _NT - 1)
    def _fin():
        se_ref[...] = jnp.sum(acc_ref[...], axis=1, keepdims=True)


_stream = pl.pallas_call(
    _stream_body,
    grid=(_NT,),
    in_specs=[
        pl.BlockSpec((BATCH, NPC_DIM), lambda i: (0, 0)),
        pl.BlockSpec((_TN, NPC_DIM), lambda i: (i, 0)),
    ],
    out_specs=[
        pl.BlockSpec((BATCH, NPC_DIM), lambda i: (0, 0)),
        pl.BlockSpec((BATCH, 1), lambda i: (0, 0)),
    ],
    out_shape=[
        jax.ShapeDtypeStruct((BATCH, NPC_DIM), jnp.float32),
        jax.ShapeDtypeStruct((BATCH, 1), jnp.float32),
    ],
    scratch_shapes=[pltpu.VMEM((BATCH, NPC_DIM), jnp.bfloat16),
                    pltpu.VMEM((BATCH, 128), jnp.float32)],
    compiler_params=pltpu.CompilerParams(
        dimension_semantics=("arbitrary",),
    ),
)


def _loss_body(znt_ref, se_ref, flg_ref, rs_ref, rl_ref, rn_ref,
               inst_ref, anch_ref):
    znt = znt_ref[...]
    inv = 1.0 / se_ref[...]
    p_self = jnp.exp2(jnp.sum(znt * rs_ref[...], axis=1, keepdims=True)) * inv
    p_ln = jnp.exp2(jnp.sum(znt * rl_ref[...], axis=1, keepdims=True)) * inv
    p_nb = jnp.exp2(jnp.sum(znt * rn_ref[...], axis=1, keepdims=True)) * inv
    flg = flg_ref[...]
    inst_terms = jnp.log(p_self + p_ln + CONST)
    anch_terms = jnp.log(p_self + p_nb + p_ln + CONST)
    scale = -2.0 / BATCH
    inst = jnp.sum(jnp.where(flg < 0, inst_terms, 0.0)) * scale
    anch = jnp.sum(jnp.where(flg >= 0, anch_terms, 0.0)) * scale
    inst_ref[...] = inst[None, None]
    anch_ref[...] = anch[None, None]


_loss = pl.pallas_call(
    _loss_body,
    out_shape=[
        jax.ShapeDtypeStruct((1, 1), jnp.float32),
        jax.ShapeDtypeStruct((1, 1), jnp.float32),
    ],
)


def kernel(zp, index, local_neighbor_indices, memory, flag, neighbors):
    flags, rows_self, rows_ln, rows_nb = _get_sc_gather()(
        index, local_neighbor_indices, memory, flag, neighbors)
    znt, se = _stream(zp, memory)
    inst, anch = _loss(znt, se, flags.reshape(BATCH, 1),
                       rows_self, rows_ln, rows_nb)
    return (inst[0, 0], anch[0, 0])


# speedup vs baseline: 1.0662x; 1.0662x over previous
import functools

import jax
import jax.numpy as jnp
from jax import lax
from jax.experimental import pallas as pl
from jax.experimental.pallas import tpu as pltpu
from jax.experimental.pallas import tpu_sc as plsc

N_SAMPLES = 100000
NPC_DIM = 128
BATCH = 4096
TEMP = 0.07
CONST = 1e-12

_NC = 2
_NS = 16
_NW = _NC * _NS
_BPW = BATCH // _NW

_TN = 2000
_NT = N_SAMPLES // _TN


@functools.cache
def _get_sc_gather():
    mesh = plsc.VectorSubcoreMesh(core_axis_name="c", subcore_axis_name="s")

    @functools.partial(
        pl.kernel,
        mesh=mesh,
        out_type=[
            jax.ShapeDtypeStruct((BATCH,), jnp.int32),
            jax.ShapeDtypeStruct((BATCH, NPC_DIM), jnp.float32),
            jax.ShapeDtypeStruct((BATCH, NPC_DIM), jnp.float32),
            jax.ShapeDtypeStruct((BATCH, NPC_DIM), jnp.float32),
        ],
        scratch_types=[
            pltpu.VMEM((_BPW,), jnp.int32),
            pltpu.VMEM((_BPW,), jnp.int32),
            pltpu.VMEM((_BPW,), jnp.int32),
            pltpu.VMEM((_BPW,), jnp.int32),
            pltpu.VMEM((_BPW, NPC_DIM), jnp.float32),
            pltpu.VMEM((_BPW, NPC_DIM), jnp.float32),
            pltpu.VMEM((_BPW, NPC_DIM), jnp.float32),
            pltpu.SemaphoreType.DMA,
        ],
    )
    def _sc_gather(index_hbm, lni_hbm, memory_hbm, flag_hbm, neighbors_hbm,
                   flags_out, rows_self_out, rows_ln_out, rows_nb_out,
                   idx_v, lni_v, nbi_v, flg_v, rs_v, rl_v, rn_v, sem):
        wid = lax.axis_index("s") * _NC + lax.axis_index("c")
        base = wid * _BPW
        pltpu.sync_copy(index_hbm.at[pl.ds(base, _BPW)], idx_v)
        pltpu.sync_copy(lni_hbm.at[pl.ds(base, _BPW)], lni_v)
        pltpu.async_copy(flag_hbm.at[idx_v], flg_v, sem).wait()
        pltpu.async_copy(neighbors_hbm.at[idx_v], nbi_v, sem).wait()
        pltpu.async_copy(memory_hbm.at[idx_v], rs_v, sem).wait()
        pltpu.async_copy(memory_hbm.at[lni_v], rl_v, sem).wait()
        pltpu.async_copy(memory_hbm.at[nbi_v], rn_v, sem).wait()
        pltpu.sync_copy(flg_v, flags_out.at[pl.ds(base, _BPW)])
        pltpu.sync_copy(rs_v, rows_self_out.at[pl.ds(base, _BPW)])
        pltpu.sync_copy(rl_v, rows_ln_out.at[pl.ds(base, _BPW)])
        pltpu.sync_copy(rn_v, rows_nb_out.at[pl.ds(base, _BPW)])

    return _sc_gather


_LOG2E = 1.4426950408889634


def _stream_body(zp_ref, mem_ref, znt_ref, se_ref, znt_bf_ref, acc_ref):
    step = pl.program_id(0)

    @pl.when(step == 0)
    def _init():
        zp = zp_ref[...]
        nrm = jnp.sqrt(jnp.sum(zp * zp, axis=1, keepdims=True))
        znt = zp * (_LOG2E / TEMP) / jnp.maximum(nrm, 1e-12)
        znt_ref[...] = znt
        znt_bf_ref[...] = znt.astype(jnp.bfloat16)
        acc_ref[...] = jnp.zeros_like(acc_ref)

    s = lax.dot_general(znt_bf_ref[...], mem_ref[...].astype(jnp.bfloat16),
                        (((1,), (1,)), ((), ())),
                        preferred_element_type=jnp.float32)
    e = jnp.exp2(s)
    t = e[:, 0:128]
    for i in range(1, _TN // 128):
        t = t + e[:, i * 128:(i + 1) * 128]
    rem = _TN - (_TN // 128) * 128
    if rem:
        tail = e[:, _TN - rem:_TN]
        t = t + jnp.pad(tail, ((0, 0), (0, 128 - rem)))
    acc_ref[...] += t

    @pl.when(step == _NT - 1)
    def _fin():
        se_ref[...] = jnp.sum(acc_ref[...], axis=1, keepdims=True)


_stream = pl.pallas_call(
    _stream_body,
    grid=(_NT,),
    in_specs=[
        pl.BlockSpec((BATCH, NPC_DIM), lambda i: (0, 0)),
        pl.BlockSpec((_TN, NPC_DIM), lambda i: (i, 0)),
    ],
    out_specs=[
        pl.BlockSpec((BATCH, NPC_DIM), lambda i: (0, 0)),
        pl.BlockSpec((BATCH, 1), lambda i: (0, 0)),
    ],
    out_shape=[
        jax.ShapeDtypeStruct((BATCH, NPC_DIM), jnp.float32),
        jax.ShapeDtypeStruct((BATCH, 1), jnp.float32),
    ],
    scratch_shapes=[pltpu.VMEM((BATCH, NPC_DIM), jnp.bfloat16),
                    pltpu.VMEM((BATCH, 128), jnp.float32)],
    compiler_params=pltpu.CompilerParams(
        dimension_semantics=("arbitrary",),
    ),
)


def _loss_body(znt_ref, se_ref, flg_ref, rs_ref, rl_ref, rn_ref,
               inst_ref, anch_ref):
    znt = znt_ref[...]
    inv = 1.0 / se_ref[...]
    p_self = jnp.exp2(jnp.sum(znt * rs_ref[...], axis=1, keepdims=True)) * inv
    p_ln = jnp.exp2(jnp.sum(znt * rl_ref[...], axis=1, keepdims=True)) * inv
    p_nb = jnp.exp2(jnp.sum(znt * rn_ref[...], axis=1, keepdims=True)) * inv
    flg = flg_ref[...]
    inst_terms = jnp.log(p_self + p_ln + CONST)
    anch_terms = jnp.log(p_self + p_nb + p_ln + CONST)
    scale = -2.0 / BATCH
    inst = jnp.sum(jnp.where(flg < 0, inst_terms, 0.0)) * scale
    anch = jnp.sum(jnp.where(flg >= 0, anch_terms, 0.0)) * scale
    inst_ref[...] = inst[None, None]
    anch_ref[...] = anch[None, None]


_loss = pl.pallas_call(
    _loss_body,
    out_shape=[
        jax.ShapeDtypeStruct((1, 1), jnp.float32),
        jax.ShapeDtypeStruct((1, 1), jnp.float32),
    ],
)


def kernel(zp, index, local_neighbor_indices, memory, flag, neighbors):
    flags, rows_self, rows_ln, rows_nb = _get_sc_gather()(
        index, local_neighbor_indices, memory, flag, neighbors)
    znt, se = _stream(zp, memory)
    inst, anch = _loss(znt, se, flags.reshape(BATCH, 1),
                       rows_self, rows_ln, rows_nb)
    return (inst[0, 0], anch[0, 0])
